# Initial kernel scaffold; baseline (speedup 1.0000x reference)
import jax
import jax.numpy as jnp
from jax.experimental import pallas as pl


def kernel(x, edge_index, W1, att_src1, att_dst1, b1, W2, att_src2, att_dst2, b2):
    raise NotImplementedError("write your pallas kernel here")


# speedup vs baseline: 3.4598x; 3.4598x over previous
import functools

import jax
import jax.numpy as jnp
from jax.experimental import pallas as pl
from jax.experimental.pallas import tpu as pltpu

_N = 10000
_E = 320000
_F_IN = 128
_HID = 128
_HEADS = 8
_NCLS = 3

_ROW_BLK = 400


def _l1_dense_kernel(x_ref, w_ref, asrc_ref, adst_ref, h_ref, as_ref, ad_ref):
    h = jnp.dot(x_ref[...], w_ref[...], preferred_element_type=jnp.float32)
    h_ref[...] = h
    r = h.shape[0]
    h3 = h.reshape(r, _HEADS, _HID)
    as_ref[...] = jnp.sum(h3 * asrc_ref[...][None, :, :], axis=-1)
    ad_ref[...] = jnp.sum(h3 * adst_ref[...][None, :, :], axis=-1)


def _l1_dense(x, W1, att_src1, att_dst1):
    n = x.shape[0]
    grid = (n // _ROW_BLK,)
    return pl.pallas_call(
        _l1_dense_kernel,
        grid=grid,
        in_specs=[
            pl.BlockSpec((_ROW_BLK, _F_IN), lambda i: (i, 0)),
            pl.BlockSpec((_F_IN, _HEADS * _HID), lambda i: (0, 0)),
            pl.BlockSpec((_HEADS, _HID), lambda i: (0, 0)),
            pl.BlockSpec((_HEADS, _HID), lambda i: (0, 0)),
        ],
        out_specs=[
            pl.BlockSpec((_ROW_BLK, _HEADS * _HID), lambda i: (i, 0)),
            pl.BlockSpec((_ROW_BLK, _HEADS), lambda i: (i, 0)),
            pl.BlockSpec((_ROW_BLK, _HEADS), lambda i: (i, 0)),
        ],
        out_shape=[
            jax.ShapeDtypeStruct((n, _HEADS * _HID), jnp.float32),
            jax.ShapeDtypeStruct((n, _HEADS), jnp.float32),
            jax.ShapeDtypeStruct((n, _HEADS), jnp.float32),
        ],
    )(x, W1, att_src1, att_dst1)


def _l2_dense_kernel(h_ref, w_ref, b1_ref, asrc_ref, adst_ref, g_ref, as_ref, ad_ref):
    t = h_ref[...] + b1_ref[...][None, :]
    g = jnp.where(t > 0, t, jnp.exp(jnp.minimum(t, 0.0)) - 1.0)
    z = jnp.dot(g, w_ref[...], preferred_element_type=jnp.float32)
    g_ref[...] = z
    as_ref[...] = jnp.sum(z * asrc_ref[...], axis=-1, keepdims=True)
    ad_ref[...] = jnp.sum(z * adst_ref[...], axis=-1, keepdims=True)


def _l2_dense(h, W2, b1, att_src2, att_dst2):
    n = h.shape[0]
    grid = (n // _ROW_BLK,)
    return pl.pallas_call(
        _l2_dense_kernel,
        grid=grid,
        in_specs=[
            pl.BlockSpec((_ROW_BLK, _HEADS * _HID), lambda i: (i, 0)),
            pl.BlockSpec((_HEADS * _HID, _NCLS), lambda i: (0, 0)),
            pl.BlockSpec((_HEADS * _HID,), lambda i: (0,)),
            pl.BlockSpec((1, _NCLS), lambda i: (0, 0)),
            pl.BlockSpec((1, _NCLS), lambda i: (0, 0)),
        ],
        out_specs=[
            pl.BlockSpec((_ROW_BLK, _NCLS), lambda i: (i, 0)),
            pl.BlockSpec((_ROW_BLK, 1), lambda i: (i, 0)),
            pl.BlockSpec((_ROW_BLK, 1), lambda i: (i, 0)),
        ],
        out_shape=[
            jax.ShapeDtypeStruct((n, _NCLS), jnp.float32),
            jax.ShapeDtypeStruct((n, 1), jnp.float32),
            jax.ShapeDtypeStruct((n, 1), jnp.float32),
        ],
    )(h, W2, b1, att_src2, att_dst2)


def _lsm_kernel(z_ref, b2_ref, o_ref):
    z = z_ref[...] + b2_ref[...][None, :]
    m = jnp.max(z, axis=-1, keepdims=True)
    s = jnp.log(jnp.sum(jnp.exp(z - m), axis=-1, keepdims=True))
    o_ref[...] = z - m - s


def _log_softmax(z, b2):
    n = z.shape[0]
    return pl.pallas_call(
        _lsm_kernel,
        grid=(n // _ROW_BLK,),
        in_specs=[
            pl.BlockSpec((_ROW_BLK, _NCLS), lambda i: (i, 0)),
            pl.BlockSpec((_NCLS,), lambda i: (0,)),
        ],
        out_specs=pl.BlockSpec((_ROW_BLK, _NCLS), lambda i: (i, 0)),
        out_shape=jax.ShapeDtypeStruct((n, _NCLS), jnp.float32),
    )(z, b2)


def _edge_softmax_agg(h, alpha_src, alpha_dst, src, dst, n, heads, ch):
    e = alpha_src[src] + alpha_dst[dst]
    e = jnp.where(e >= 0, e, 0.2 * e)
    ex = jnp.exp(e)
    denom = jax.ops.segment_sum(ex, dst, num_segments=n)
    alpha = ex / (denom[dst] + 1e-16)
    msg = h[src].reshape(-1, heads, ch) * alpha[:, :, None]
    out = jax.ops.segment_sum(msg.reshape(-1, heads * ch), dst, num_segments=n)
    return out


def kernel(x, edge_index, W1, att_src1, att_dst1, b1, W2, att_src2, att_dst2, b2):
    n = x.shape[0]
    loop = jnp.arange(n, dtype=edge_index.dtype)
    src = jnp.concatenate([edge_index[0], loop])
    dst = jnp.concatenate([edge_index[1], loop])

    h1, as1, ad1 = _l1_dense(x, W1, att_src1, att_dst1)
    agg1 = _edge_softmax_agg(h1, as1, ad1, src, dst, n, _HEADS, _HID)

    z2, as2, ad2 = _l2_dense(agg1, W2, b1, att_src2, att_dst2)
    agg2 = _edge_softmax_agg(z2, as2, ad2, src, dst, n, 1, _NCLS)

    return _log_softmax(agg2, b2)
